# Initial kernel scaffold; baseline (speedup 1.0000x reference)
import jax
import jax.numpy as jnp
from jax.experimental import pallas as pl


def kernel(x, Wg, W1, b1, W2, b2):
    raise NotImplementedError("write your pallas kernel here")


# speedup vs baseline: 5.1035x; 5.1035x over previous
import functools

import jax
import jax.numpy as jnp
from jax.experimental import pallas as pl

N_EXP = 16
N_SLOTS = 2 * N_EXP
TB = 512
FB = 1024


def _router_body(tok_ref, wg_ref, meta_ref, imp_ref):
    i = pl.program_id(0)
    tok = tok_ref[...]
    wg = wg_ref[...]
    logits = jax.lax.dot_general(tok, wg, (((1,), (1,)), ((), ())),
                                 preferred_element_type=jnp.float32)
    m = jnp.max(logits, axis=1, keepdims=True)
    p = jnp.exp(logits - m)
    probs = p / jnp.sum(p, axis=1, keepdims=True)
    eio = jax.lax.broadcasted_iota(jnp.int32, (TB, N_EXP), 1).astype(jnp.float32)
    mx1 = jnp.max(probs, axis=1, keepdims=True)
    t1 = jnp.min(jnp.where(probs >= mx1, eio, float(N_EXP)), axis=1, keepdims=True)
    probs2 = jnp.where(eio == t1, -1.0, probs)
    mx2 = jnp.max(probs2, axis=1, keepdims=True)
    t2 = jnp.min(jnp.where(probs2 >= mx2, eio, float(N_EXP)), axis=1, keepdims=True)
    cio = jax.lax.broadcasted_iota(jnp.int32, (TB, 8), 1).astype(jnp.float32)
    meta = (jnp.where(cio == 0.0, t1, 0.0) + jnp.where(cio == 1.0, t2, 0.0)
            + jnp.where(cio == 2.0, mx1, 0.0) + jnp.where(cio == 3.0, mx2, 0.0))
    meta_ref[...] = meta
    pi = jnp.sum(probs, axis=0, keepdims=True)

    @pl.when(i == 0)
    def _():
        imp_ref[...] = pi

    @pl.when(i != 0)
    def _():
        imp_ref[...] += pi


def _cumsum_axis0(a, s):
    k = 1
    while k < s:
        pad = jnp.zeros((k, a.shape[1]), a.dtype)
        a = a + jnp.concatenate([pad, a[:s - k, :]], axis=0)
        k *= 2
    return a


def _routing_body(meta_ref, tok_ref, imp_ref, wmat_ref, win_ref, loss_ref):
    s = meta_ref.shape[0]
    meta = meta_ref[...]
    t1 = meta[:, 0:1]
    t2 = meta[:, 1:2]
    g1 = meta[:, 2:3]
    g2 = meta[:, 3:4]
    eio = jax.lax.broadcasted_iota(jnp.int32, (s, N_EXP), 1).astype(jnp.float32)
    oh1 = (eio == t1).astype(jnp.float32)
    oh2 = (eio == t2).astype(jnp.float32)
    hits = oh1 + oh2
    incl = _cumsum_axis0(hits, s)
    excl = incl - hits
    counts = incl[s - 1:s, :]
    mp1 = jnp.max(jnp.where(eio < t1, counts, 0.0), axis=1, keepdims=True)
    mp2 = jnp.max(jnp.where(eio < t2, counts, 0.0), axis=1, keepdims=True)
    ex1 = jnp.sum(oh1 * excl, axis=1, keepdims=True)
    ex2 = jnp.sum(oh2 * excl, axis=1, keepdims=True)
    s1 = 2.0 * t1 + (ex1 >= mp1).astype(jnp.float32)
    s2 = 2.0 * t2 + (ex2 >= mp2).astype(jnp.float32)
    sio = jax.lax.broadcasted_iota(jnp.int32, (s, N_SLOTS), 1).astype(jnp.float32)
    tio = jax.lax.broadcasted_iota(jnp.int32, (s, N_SLOTS), 0).astype(jnp.float32)
    gsum = g1 + g2
    wmat_ref[...] = (jnp.where(sio == s1, g1, 0.0)
                     + jnp.where(sio == s2, g2, 0.0)) / gsum
    f1 = jnp.where(sio == s1, 2.0 * tio, -1.0)
    f2 = jnp.where(sio == s2, 2.0 * tio + 1.0, -1.0)
    wflat = jnp.max(jnp.maximum(f1, f2), axis=0, keepdims=True)
    wtok = jnp.floor(wflat * 0.5)
    gm = (tio == wtok).astype(jnp.float32)
    winners = jax.lax.dot_general(gm, tok_ref[...], (((0,), (0,)), ((), ())),
                                  preferred_element_type=jnp.float32)
    win_ref[...] = winners
    imp = imp_ref[...]
    loss_ref[...] = jnp.sum(imp * counts, axis=1, keepdims=True) * (
        float(N_EXP) / (float(s) * float(s)))


def _ffn_body(win_ref, w1_ref, b1_ref, w2_ref, b2_ref, out_ref):
    c = pl.program_id(1)
    wt = win_ref[0]
    h = jax.lax.dot_general(wt, w1_ref[0], (((1,), (1,)), ((), ())),
                            preferred_element_type=jnp.float32)
    h = h + b1_ref[0]
    h = 0.5 * h * (1.0 + jax.lax.erf(h * (2.0 ** -0.5)))
    o = jax.lax.dot_general(h, w2_ref[0], (((1,), (1,)), ((), ())),
                            preferred_element_type=jnp.float32)

    @pl.when(c == 0)
    def _():
        out_ref[0] = o + b2_ref[0]

    @pl.when(c != 0)
    def _():
        out_ref[0] += o


def _combine_body(wmat_ref, tab_ref, y_ref):
    y_ref[...] = jax.lax.dot_general(wmat_ref[...], tab_ref[...],
                                     (((1,), (0,)), ((), ())),
                                     preferred_element_type=jnp.float32)


def kernel(x, Wg, W1, b1, W2, b2):
    B, T, D = x.shape
    S = B * T
    D_FF = W1.shape[1]
    tokens = x.reshape(S, D)
    nb = S // TB

    meta, imp = pl.pallas_call(
        _router_body,
        grid=(nb,),
        in_specs=[
            pl.BlockSpec((TB, D), lambda i: (i, 0)),
            pl.BlockSpec((N_EXP, D), lambda i: (0, 0)),
        ],
        out_specs=[
            pl.BlockSpec((TB, 8), lambda i: (i, 0)),
            pl.BlockSpec((1, N_EXP), lambda i: (0, 0)),
        ],
        out_shape=[
            jax.ShapeDtypeStruct((S, 8), jnp.float32),
            jax.ShapeDtypeStruct((1, N_EXP), jnp.float32),
        ],
    )(tokens, Wg)

    wmat, winners, loss = pl.pallas_call(
        _routing_body,
        in_specs=[
            pl.BlockSpec((S, 8), lambda: (0, 0)),
            pl.BlockSpec((S, D), lambda: (0, 0)),
            pl.BlockSpec((1, N_EXP), lambda: (0, 0)),
        ],
        out_specs=[
            pl.BlockSpec((S, N_SLOTS), lambda: (0, 0)),
            pl.BlockSpec((N_SLOTS, D), lambda: (0, 0)),
            pl.BlockSpec((1, 1), lambda: (0, 0)),
        ],
        out_shape=[
            jax.ShapeDtypeStruct((S, N_SLOTS), jnp.float32),
            jax.ShapeDtypeStruct((N_SLOTS, D), jnp.float32),
            jax.ShapeDtypeStruct((1, 1), jnp.float32),
        ],
    )(meta, tokens, imp)

    nc = D_FF // FB
    table = pl.pallas_call(
        _ffn_body,
        grid=(N_EXP, nc),
        in_specs=[
            pl.BlockSpec((1, 2, D), lambda e, c: (e, 0, 0)),
            pl.BlockSpec((1, FB, D), lambda e, c: (e, c, 0)),
            pl.BlockSpec((1, 1, FB), lambda e, c: (e, 0, c)),
            pl.BlockSpec((1, D, FB), lambda e, c: (e, 0, c)),
            pl.BlockSpec((1, 1, D), lambda e, c: (e, 0, 0)),
        ],
        out_specs=pl.BlockSpec((1, 2, D), lambda e, c: (e, 0, 0)),
        out_shape=jax.ShapeDtypeStruct((N_EXP, 2, D), jnp.float32),
    )(winners.reshape(N_EXP, 2, D), W1, b1.reshape(N_EXP, 1, D_FF),
      W2, b2.reshape(N_EXP, 1, D))
    table = table.reshape(N_SLOTS, D)

    y = pl.pallas_call(
        _combine_body,
        grid=(nb,),
        in_specs=[
            pl.BlockSpec((TB, N_SLOTS), lambda i: (i, 0)),
            pl.BlockSpec((N_SLOTS, D), lambda i: (0, 0)),
        ],
        out_specs=pl.BlockSpec((TB, D), lambda i: (i, 0)),
        out_shape=jax.ShapeDtypeStruct((S, D), jnp.float32),
    )(wmat, table)

    return y.reshape(B, T, D), loss[0, 0]


# speedup vs baseline: 5.4032x; 1.0587x over previous
import jax
import jax.numpy as jnp
from jax.experimental import pallas as pl

N_EXP = 16
N_SLOTS = 2 * N_EXP
TB = 512
FB = 1024


def _cumsum_axis0(a, s):
    k = 1
    while k < s:
        pad = jnp.zeros((k, a.shape[1]), a.dtype)
        a = a + jnp.concatenate([pad, a[:s - k, :]], axis=0)
        k *= 2
    return a


def _route_body(tok_ref, wg_ref, wmat_ref, win_ref, loss_ref):
    s = tok_ref.shape[0]
    tok = tok_ref[...]
    logits = jax.lax.dot_general(tok, wg_ref[...], (((1,), (1,)), ((), ())),
                                 preferred_element_type=jnp.float32)
    m = jnp.max(logits, axis=1, keepdims=True)
    p = jnp.exp(logits - m)
    probs = p / jnp.sum(p, axis=1, keepdims=True)
    eio = jax.lax.broadcasted_iota(jnp.int32, (s, N_EXP), 1).astype(jnp.float32)
    mx1 = jnp.max(probs, axis=1, keepdims=True)
    t1 = jnp.min(jnp.where(probs >= mx1, eio, float(N_EXP)), axis=1,
                 keepdims=True)
    probs2 = jnp.where(eio == t1, -1.0, probs)
    mx2 = jnp.max(probs2, axis=1, keepdims=True)
    t2 = jnp.min(jnp.where(probs2 >= mx2, eio, float(N_EXP)), axis=1,
                 keepdims=True)
    g1, g2 = mx1, mx2
    imp = jnp.sum(probs, axis=0, keepdims=True)

    oh1 = (eio == t1).astype(jnp.float32)
    oh2 = (eio == t2).astype(jnp.float32)
    hits = oh1 + oh2
    incl = _cumsum_axis0(hits, s)
    excl = incl - hits
    counts = incl[s - 1:s, :]
    mp1 = jnp.max(jnp.where(eio < t1, counts, 0.0), axis=1, keepdims=True)
    mp2 = jnp.max(jnp.where(eio < t2, counts, 0.0), axis=1, keepdims=True)
    ex1 = jnp.sum(oh1 * excl, axis=1, keepdims=True)
    ex2 = jnp.sum(oh2 * excl, axis=1, keepdims=True)
    s1 = 2.0 * t1 + (ex1 >= mp1).astype(jnp.float32)
    s2 = 2.0 * t2 + (ex2 >= mp2).astype(jnp.float32)
    sio = jax.lax.broadcasted_iota(jnp.int32, (s, N_SLOTS), 1).astype(jnp.float32)
    tio = jax.lax.broadcasted_iota(jnp.int32, (s, N_SLOTS), 0).astype(jnp.float32)
    gsum = g1 + g2
    wmat_ref[...] = (jnp.where(sio == s1, g1, 0.0)
                     + jnp.where(sio == s2, g2, 0.0)) / gsum
    f1 = jnp.where(sio == s1, 2.0 * tio, -1.0)
    f2 = jnp.where(sio == s2, 2.0 * tio + 1.0, -1.0)
    wflat = jnp.max(jnp.maximum(f1, f2), axis=0, keepdims=True)
    wtok = jnp.floor(wflat * 0.5)
    gm = (tio == wtok).astype(jnp.float32)
    winners = jax.lax.dot_general(gm, tok, (((0,), (0,)), ((), ())),
                                  preferred_element_type=jnp.float32)
    win_ref[...] = winners
    loss_ref[...] = jnp.sum(imp * counts, axis=1, keepdims=True) * (
        float(N_EXP) / (float(s) * float(s)))


def _ffn_body(win_ref, w1_ref, b1_ref, w2_ref, b2_ref, out_ref):
    c = pl.program_id(1)
    wt = win_ref[0]
    h = jax.lax.dot_general(wt, w1_ref[0], (((1,), (1,)), ((), ())),
                            preferred_element_type=jnp.float32)
    h = h + b1_ref[0]
    h = 0.5 * h * (1.0 + jax.lax.erf(h * (2.0 ** -0.5)))
    o = jax.lax.dot_general(h, w2_ref[0], (((1,), (1,)), ((), ())),
                            preferred_element_type=jnp.float32)

    @pl.when(c == 0)
    def _():
        out_ref[0] = o + b2_ref[0]

    @pl.when(c != 0)
    def _():
        out_ref[0] += o


def _combine_body(wmat_ref, tab_ref, y_ref):
    y_ref[...] = jax.lax.dot_general(wmat_ref[...], tab_ref[...],
                                     (((1,), (0,)), ((), ())),
                                     preferred_element_type=jnp.float32)


def kernel(x, Wg, W1, b1, W2, b2):
    B, T, D = x.shape
    S = B * T
    D_FF = W1.shape[1]
    tokens = x.reshape(S, D)
    nb = S // TB

    wmat, winners, loss = pl.pallas_call(
        _route_body,
        in_specs=[
            pl.BlockSpec((S, D), lambda: (0, 0)),
            pl.BlockSpec((N_EXP, D), lambda: (0, 0)),
        ],
        out_specs=[
            pl.BlockSpec((S, N_SLOTS), lambda: (0, 0)),
            pl.BlockSpec((N_SLOTS, D), lambda: (0, 0)),
            pl.BlockSpec((1, 1), lambda: (0, 0)),
        ],
        out_shape=[
            jax.ShapeDtypeStruct((S, N_SLOTS), jnp.float32),
            jax.ShapeDtypeStruct((N_SLOTS, D), jnp.float32),
            jax.ShapeDtypeStruct((1, 1), jnp.float32),
        ],
    )(tokens, Wg)

    nc = D_FF // FB
    table = pl.pallas_call(
        _ffn_body,
        grid=(N_EXP, nc),
        in_specs=[
            pl.BlockSpec((1, 2, D), lambda e, c: (e, 0, 0)),
            pl.BlockSpec((1, FB, D), lambda e, c: (e, c, 0)),
            pl.BlockSpec((1, 1, FB), lambda e, c: (e, 0, c)),
            pl.BlockSpec((1, D, FB), lambda e, c: (e, 0, c)),
            pl.BlockSpec((1, 1, D), lambda e, c: (e, 0, 0)),
        ],
        out_specs=pl.BlockSpec((1, 2, D), lambda e, c: (e, 0, 0)),
        out_shape=jax.ShapeDtypeStruct((N_EXP, 2, D), jnp.float32),
    )(winners.reshape(N_EXP, 2, D), W1, b1.reshape(N_EXP, 1, D_FF),
      W2, b2.reshape(N_EXP, 1, D))
    table = table.reshape(N_SLOTS, D)

    y = pl.pallas_call(
        _combine_body,
        grid=(nb,),
        in_specs=[
            pl.BlockSpec((TB, N_SLOTS), lambda i: (i, 0)),
            pl.BlockSpec((N_SLOTS, D), lambda i: (0, 0)),
        ],
        out_specs=pl.BlockSpec((TB, D), lambda i: (i, 0)),
        out_shape=jax.ShapeDtypeStruct((S, D), jnp.float32),
    )(wmat, table)

    return y.reshape(B, T, D), loss[0, 0]
